# baseline (speedup 1.0000x reference)
import functools

import jax
import jax.numpy as jnp
from jax import lax
from jax.experimental import pallas as pl
from jax.experimental.pallas import tpu as pltpu
from jax.experimental.pallas import tpu_sc as plsc

_K = 8192
_D = 256
_N = 4096
_BETA = 0.25

_NB = 512
_KB = 1024


def _argmin_body(x_ref, e_ref, out_ref, minval, minidx):
    j = pl.program_id(1)
    x = x_ref[...]
    e = e_ref[...]
    znorm = jnp.sum(x * x, axis=1, keepdims=True)
    enorm = jnp.sum(e * e, axis=1, keepdims=True)
    scores = lax.dot_general(
        x, e, (((1,), (1,)), ((), ())),
        preferred_element_type=jnp.float32)
    dist = (znorm + enorm.reshape(1, _KB)) - 2.0 * scores
    bmin = jnp.min(dist, axis=1, keepdims=True)
    iota = lax.broadcasted_iota(jnp.int32, (_NB, _KB), 1)
    bidx = jnp.min(jnp.where(dist == bmin, iota, jnp.int32(2**30)),
                   axis=1, keepdims=True) + j * _KB

    @pl.when(j == 0)
    def _():
        minval[...] = bmin
        minidx[...] = bidx

    @pl.when(j > 0)
    def _():
        upd = bmin < minval[...]
        minidx[...] = jnp.where(upd, bidx, minidx[...])
        minval[...] = jnp.where(upd, bmin, minval[...])

    @pl.when(j == pl.num_programs(1) - 1)
    def _():
        out_ref[...] = minidx[...]


def _argmin_call(flat, emb, interpret=False):
    return pl.pallas_call(
        _argmin_body,
        grid=(_N // _NB, _K // _KB),
        in_specs=[
            pl.BlockSpec((_NB, _D), lambda i, j: (i, 0)),
            pl.BlockSpec((_KB, _D), lambda i, j: (j, 0)),
        ],
        out_specs=pl.BlockSpec((_NB, 1), lambda i, j: (i, 0)),
        out_shape=jax.ShapeDtypeStruct((_N, 1), jnp.int32),
        scratch_shapes=[
            pltpu.VMEM((_NB, 1), jnp.float32),
            pltpu.VMEM((_NB, 1), jnp.int32),
        ],
        compiler_params=pltpu.CompilerParams(
            dimension_semantics=("parallel", "arbitrary")),
        interpret=interpret,
    )(flat, emb)


def _make_gather():
    info = plsc.get_sparse_core_info()
    nc, ns = info.num_cores, info.num_subcores
    nw = nc * ns
    b_per_w = _N // nw
    mesh = plsc.VectorSubcoreMesh(core_axis_name="c", subcore_axis_name="s")

    @functools.partial(
        pl.kernel, mesh=mesh,
        out_type=jax.ShapeDtypeStruct((_N, _D), jnp.float32),
        scratch_types=[
            pltpu.VMEM((b_per_w,), jnp.int32),
            pltpu.VMEM((b_per_w, _D), jnp.float32),
            pltpu.SemaphoreType.DMA,
        ],
    )
    def gather(table_hbm, idx_hbm, out_hbm, idx_v, rows_v, sem):
        wid = lax.axis_index("s") * nc + lax.axis_index("c")
        base = wid * b_per_w
        pltpu.sync_copy(idx_hbm.at[pl.ds(base, b_per_w)], idx_v)
        pltpu.async_copy(table_hbm.at[idx_v], rows_v, sem).wait()
        pltpu.sync_copy(rows_v, out_hbm.at[pl.ds(base, b_per_w)])

    return gather


def _st_loss_body(x_ref, q_ref, o_ref, s_ref):
    x = x_ref[...]
    q = q_ref[...]
    d = q - x
    o_ref[...] = x + d
    s_ref[0, 0] = jnp.sum(d * d)


def _st_loss_call(flat, q, interpret=False):
    return pl.pallas_call(
        _st_loss_body,
        out_shape=(
            jax.ShapeDtypeStruct((_N, _D), jnp.float32),
            jax.ShapeDtypeStruct((1, 1), jnp.float32),
        ),
        out_specs=(
            pl.BlockSpec((_N, _D), lambda: (0, 0)),
            pl.BlockSpec(memory_space=pltpu.SMEM),
        ),
        interpret=interpret,
    )(flat, q)


def kernel(latents, validation, embedding_weight):
    lat = jnp.transpose(latents, (0, 2, 3, 1))
    flat = lat.reshape(_N, _D)
    inds = _argmin_call(flat, embedding_weight)
    q = _make_gather()(embedding_weight, inds.reshape(_N))
    out_flat, ssum = _st_loss_call(flat, q)
    m = ssum[0, 0] / jnp.float32(_N * _D)
    vq_loss = m * jnp.float32(_BETA) + m
    out = out_flat.reshape(4, 32, 32, _D).transpose(0, 3, 1, 2)
    return out, vq_loss


# speedup vs baseline: 1.3032x; 1.3032x over previous
import functools

import jax
import jax.numpy as jnp
from jax import lax
from jax.experimental import pallas as pl
from jax.experimental.pallas import tpu as pltpu
from jax.experimental.pallas import tpu_sc as plsc

_K = 8192
_D = 256
_N = 4096
_BETA = 0.25

_NB = 512
_KB = 2048


def _argmin_body(x_ref, e_ref, enorm_ref, out_ref):
    x = x_ref[...]
    znorm = jnp.sum(x * x, axis=1, keepdims=True)
    x2 = x + x
    runmin = None
    runidx = None
    for c in range(_K // _KB):
        ec = e_ref[pl.ds(c * _KB, _KB), :]
        s2 = lax.dot_general(
            x2, ec, (((1,), (1,)), ((), ())),
            preferred_element_type=jnp.float32)
        dist = (znorm + enorm_ref[0:1, pl.ds(c * _KB, _KB)]) - s2
        bmin = jnp.min(dist, axis=1, keepdims=True)
        iota = lax.broadcasted_iota(jnp.int32, (_NB, _KB), 1).astype(jnp.float32)
        bidx = jnp.min(jnp.where(dist == bmin, iota, jnp.float32(65536.0)),
                       axis=1, keepdims=True) + jnp.float32(c * _KB)
        if c == 0:
            runmin, runidx = bmin, bidx
        else:
            upd = bmin < runmin
            runidx = jnp.where(upd, bidx, runidx)
            runmin = jnp.where(upd, bmin, runmin)
    out_ref[...] = runidx.astype(jnp.int32)


def _argmin_call(flat, emb, interpret=False):
    enorm = jnp.sum(emb ** 2, axis=1).reshape(1, _K)
    return pl.pallas_call(
        _argmin_body,
        grid=(_N // _NB,),
        in_specs=[
            pl.BlockSpec((_NB, _D), lambda i: (i, 0)),
            pl.BlockSpec((_K, _D), lambda i: (0, 0)),
            pl.BlockSpec((1, _K), lambda i: (0, 0)),
        ],
        out_specs=pl.BlockSpec((_NB, 1), lambda i: (i, 0)),
        out_shape=jax.ShapeDtypeStruct((_N, 1), jnp.int32),
        compiler_params=pltpu.CompilerParams(
            dimension_semantics=("arbitrary",)),
        interpret=interpret,
    )(flat, emb, enorm)


def _make_gather():
    info = plsc.get_sparse_core_info()
    nc, ns = info.num_cores, info.num_subcores
    nw = nc * ns
    b_per_w = _N // nw
    mesh = plsc.VectorSubcoreMesh(core_axis_name="c", subcore_axis_name="s")

    @functools.partial(
        pl.kernel, mesh=mesh,
        out_type=jax.ShapeDtypeStruct((_N, _D), jnp.float32),
        scratch_types=[
            pltpu.VMEM((b_per_w,), jnp.int32),
            pltpu.VMEM((b_per_w, _D), jnp.float32),
            pltpu.SemaphoreType.DMA,
        ],
    )
    def gather(table_hbm, idx_hbm, out_hbm, idx_v, rows_v, sem):
        wid = lax.axis_index("s") * nc + lax.axis_index("c")
        base = wid * b_per_w
        pltpu.sync_copy(idx_hbm.at[pl.ds(base, b_per_w)], idx_v)
        pltpu.async_copy(table_hbm.at[idx_v], rows_v, sem).wait()
        pltpu.sync_copy(rows_v, out_hbm.at[pl.ds(base, b_per_w)])

    return gather


def _st_loss_body(x_ref, q_ref, o_ref, s_ref):
    x = x_ref[...]
    q = q_ref[...]
    d = q - x
    o_ref[...] = x + d
    s_ref[0, 0] = jnp.sum(d * d)


def _st_loss_call(flat, q, interpret=False):
    return pl.pallas_call(
        _st_loss_body,
        out_shape=(
            jax.ShapeDtypeStruct((_N, _D), jnp.float32),
            jax.ShapeDtypeStruct((1, 1), jnp.float32),
        ),
        out_specs=(
            pl.BlockSpec((_N, _D), lambda: (0, 0)),
            pl.BlockSpec(memory_space=pltpu.SMEM),
        ),
        interpret=interpret,
    )(flat, q)


def kernel(latents, validation, embedding_weight):
    lat = jnp.transpose(latents, (0, 2, 3, 1))
    flat = lat.reshape(_N, _D)
    inds = _argmin_call(flat, embedding_weight)
    q = _make_gather()(embedding_weight, inds.reshape(_N))
    out_flat, ssum = _st_loss_call(flat, q)
    m = ssum[0, 0] / jnp.float32(_N * _D)
    vq_loss = m * jnp.float32(_BETA) + m
    out = out_flat.reshape(4, 32, 32, _D).transpose(0, 3, 1, 2)
    return out, vq_loss
